# baseline (speedup 1.0000x reference)
import functools

import jax
import jax.numpy as jnp
from jax import lax
from jax.experimental import pallas as pl
from jax.experimental.pallas import tpu as pltpu
from jax.experimental.pallas import tpu_sc as plsc

GAMMA = 0.1
EPSILON = 0.1

NC = 2
NS = 16
K = 120
NI = 6
NG = 3
ZR = 80


@functools.cache
def _sc_mesh():
    return plsc.VectorSubcoreMesh(core_axis_name="core",
                                  subcore_axis_name="subcore",
                                  num_cores=NC, num_subcores=NS)


def _deg_body(npad, nch_deg, idx_hbm, ones_hbm, zeros_hbm, degp_hbm,
              cols_v, ones_v, zbuf, deg_sh):
    stripe = npad // NS
    c = lax.axis_index("core")
    s = lax.axis_index("subcore")
    pltpu.sync_copy(zeros_hbm, zbuf)
    pltpu.sync_copy(zbuf, deg_sh.at[pl.ds(s * stripe, stripe)])
    pltpu.sync_copy(idx_hbm.at[0, s, pl.ds(c * nch_deg, nch_deg), 1], cols_v)
    pltpu.sync_copy(ones_hbm, ones_v)
    plsc.subcore_barrier()

    @pl.loop(0, nch_deg)
    def _(j):
        pltpu.sync_copy(ones_v, deg_sh.at[cols_v.at[j]], add=True)

    plsc.subcore_barrier()
    pltpu.sync_copy(deg_sh.at[pl.ds(s * stripe, stripe)], zbuf)
    pltpu.sync_copy(zbuf, degp_hbm.at[pl.ds(c * npad + s * stripe, stripe)])


def _gcn_body(npad, nch, zrows, xws_hbm, idx_hbm, zeros_hbm, gcn_hbm,
              ig, gb, acc_sh, isems, gsems, ssems):
    stripe = npad // NS
    ni = 6
    ng = len(gb)
    c = lax.axis_index("core")
    s = lax.axis_index("subcore")
    pltpu.sync_copy(zeros_hbm, gb[0].at[pl.ds(0, zrows)])

    @pl.loop(0, stripe, step=zrows)
    def _(i):
        pltpu.sync_copy(gb[0].at[pl.ds(0, zrows)],
                        acc_sh.at[pl.ds(s * stripe + i, zrows)])

    plsc.subcore_barrier()

    pltpu.async_copy(idx_hbm.at[c, s, pl.ds(0, ni)], ig[0], isems[0])

    @pl.loop(0, nch, step=2 * ni)
    def _(j):
        for gg in range(2):
            j6 = j + ni * gg
            for u in range(ni):
                t = j6 + u
                tg = (u + 1) % ng
                if u == 0:
                    pltpu.make_async_copy(idx_hbm.at[c, s, pl.ds(j6, ni)],
                                          ig[gg], isems[gg]).wait()
                if gg == 0 and u < 3:
                    @pl.when(t >= 3)
                    def _():
                        pltpu.make_async_copy(
                            gb[u % ng], acc_sh.at[ig[gg].at[0, 1]],
                            ssems[u % ng]).wait()
                else:
                    pltpu.make_async_copy(gb[u % ng],
                                          acc_sh.at[ig[gg].at[0, 1]],
                                          ssems[u % ng]).wait()
                if u == 3:
                    @pl.when(j6 + ni < nch)
                    def _():
                        pltpu.async_copy(idx_hbm.at[c, s, pl.ds(j6 + ni, ni)],
                                         ig[1 - gg], isems[1 - gg])
                iref = ig[gg].at[u - 2, 1] if u >= 2 else ig[1 - gg].at[u + 4, 1]
                irefg = ig[gg].at[u - 2, 0] if u >= 2 else ig[1 - gg].at[u + 4, 0]
                if gg == 0 and u < 2:
                    @pl.when(t >= 2)
                    def _():
                        pltpu.make_async_copy(xws_hbm.at[irefg], gb[tg],
                                              gsems[tg]).wait()
                        pltpu.async_copy(gb[tg], acc_sh.at[iref], ssems[tg],
                                         add=True)
                else:
                    pltpu.make_async_copy(xws_hbm.at[irefg], gb[tg],
                                          gsems[tg]).wait()
                    pltpu.async_copy(gb[tg], acc_sh.at[iref], ssems[tg],
                                     add=True)
                pltpu.async_copy(xws_hbm.at[ig[gg].at[u, 0]], gb[u % ng],
                                 gsems[u % ng])

    pltpu.make_async_copy(gb[(nch - 3) % ng], acc_sh.at[ig[1].at[3, 1]],
                          ssems[(nch - 3) % ng]).wait()
    for u in (4, 5):
        t = nch - 6 + u
        pltpu.make_async_copy(xws_hbm.at[ig[1].at[u, 0]], gb[t % ng],
                              gsems[t % ng]).wait()
        pltpu.sync_copy(gb[t % ng], acc_sh.at[ig[1].at[u, 1]], add=True)

    plsc.subcore_barrier()

    nz = stripe // zrows
    base = s * stripe
    for k in range(2):
        pltpu.async_copy(acc_sh.at[pl.ds(base + k * zrows, zrows)],
                         gb[k].at[pl.ds(0, zrows)], gsems[k])
    for k in range(nz):
        if 1 <= k and k + 2 <= nz - 1:
            pltpu.make_async_copy(gb[(k - 1) % ng].at[pl.ds(0, zrows)],
                                  gcn_hbm.at[c, pl.ds(base, zrows)],
                                  ssems[(k - 1) % ng]).wait()
        if k + 2 <= nz - 1:
            pltpu.async_copy(acc_sh.at[pl.ds(base + (k + 2) * zrows, zrows)],
                             gb[(k + 2) % ng].at[pl.ds(0, zrows)],
                             gsems[(k + 2) % ng])
        pltpu.make_async_copy(acc_sh.at[pl.ds(base, zrows)],
                              gb[k % ng].at[pl.ds(0, zrows)],
                              gsems[k % ng]).wait()
        pltpu.async_copy(gb[k % ng].at[pl.ds(0, zrows)],
                         gcn_hbm.at[c, pl.ds(base + k * zrows, zrows)],
                         ssems[k % ng])
    for k in (nz - 3, nz - 2, nz - 1):
        pltpu.make_async_copy(gb[k % ng].at[pl.ds(0, zrows)],
                              gcn_hbm.at[c, pl.ds(base, zrows)],
                              ssems[k % ng]).wait()


def _dense_body(x_ref, wcat_ref, degp0_ref, degp1_ref, h2_ref, xws_ref):
    xb = x_ref[...]
    m = jnp.dot(xb.astype(jnp.bfloat16), wcat_ref[...].astype(jnp.bfloat16),
                preferred_element_type=jnp.float32)
    d = xb.shape[1]
    h2_ref[...] = m[:, d:]
    deg = degp0_ref[...] + degp1_ref[...]
    dis = jnp.where(deg > 0.0, lax.rsqrt(deg), 0.0)
    xw = m[:, :d] * dis
    half = d // 2
    xws_ref[0] = xw[:, :half]
    xws_ref[1] = xw[:, half:]


def _combine_body(x_ref, h2_ref, gcn_ref, degp0_ref, degp1_ref, bias_ref,
                  o_ref):
    deg = degp0_ref[...] + degp1_ref[...]
    dis = jnp.where(deg > 0.0, lax.rsqrt(deg), 0.0)
    g = jnp.concatenate([gcn_ref[0], gcn_ref[1]], axis=1)
    h = h2_ref[...] + g * dis + bias_ref[...]
    o_ref[...] = x_ref[...] + EPSILON * jnp.tanh(h)


def kernel(x, edge_index, W, W_phi, bias):
    n, d = x.shape
    e = edge_index.shape[1]
    half = d // 2
    npad = ((n + 2 * ZR * NS - 1) // (2 * ZR * NS)) * (2 * ZR * NS)
    stripe = npad // NS
    nch = 12 * ((e + 12 * NS * K - 1) // (12 * NS * K))
    epad = NS * K * nch
    nch_deg = nch // 2

    ep = jnp.pad(edge_index.astype(jnp.int32), ((0, 0), (0, epad - e)))
    er = ep.reshape(2, NS, nch, K)
    pos = (lax.broadcasted_iota(jnp.int32, (NS, nch, K), 0) * (nch * K)
           + lax.broadcasted_iota(jnp.int32, (NS, nch, K), 1) * K
           + lax.broadcasted_iota(jnp.int32, (NS, nch, K), 2))
    rows3 = er[0]
    cols3 = jnp.where(pos < e, er[1], npad - 1)
    idx_pack = jnp.stack(
        [jnp.stack([rows3, cols3], axis=2),
         jnp.stack([rows3 + n, cols3], axis=2)], axis=0)

    ones128 = jnp.ones((K,), jnp.float32)
    zeros1 = jnp.zeros((stripe,), jnp.float32)
    zeros2 = jnp.zeros((ZR, half), jnp.float32)

    wcat = jnp.concatenate(
        [W_phi.T, (W - W.T - GAMMA * jnp.eye(d, dtype=x.dtype)).T], axis=1)

    deg_call = pl.kernel(
        functools.partial(_deg_body, npad, nch_deg),
        out_type=jax.ShapeDtypeStruct((NC * npad,), jnp.float32),
        mesh=_sc_mesh(),
        scratch_types=[
            pltpu.VMEM((nch_deg, K), jnp.int32),
            pltpu.VMEM((K,), jnp.float32),
            pltpu.VMEM((stripe,), jnp.float32),
            pltpu.VMEM_SHARED((npad,), jnp.float32),
        ],
    )
    degp = deg_call(idx_pack, ones128, zeros1)
    degp0 = degp[:npad].reshape(npad, 1)
    degp1 = degp[npad:].reshape(npad, 1)

    nb = 10
    r = n // nb
    h2, xws = pl.pallas_call(
        _dense_body,
        grid=(nb,),
        in_specs=[
            pl.BlockSpec((r, d), lambda i: (i, 0)),
            pl.BlockSpec((d, 2 * d), lambda i: (0, 0)),
            pl.BlockSpec((r, 1), lambda i: (i, 0)),
            pl.BlockSpec((r, 1), lambda i: (i, 0)),
        ],
        out_specs=[
            pl.BlockSpec((r, d), lambda i: (i, 0)),
            pl.BlockSpec((2, r, half), lambda i: (0, i, 0)),
        ],
        out_shape=[
            jax.ShapeDtypeStruct((n, d), jnp.float32),
            jax.ShapeDtypeStruct((2, n, half), jnp.float32),
        ],
    )(x, wcat, degp0, degp1)

    gcn_call = pl.kernel(
        functools.partial(_gcn_body, npad, nch, ZR),
        out_type=jax.ShapeDtypeStruct((NC, npad, half), jnp.float32),
        mesh=_sc_mesh(),
        scratch_types=[
            [pltpu.VMEM((6, 2, K), jnp.int32) for _ in range(2)],
            [pltpu.VMEM((K, half), jnp.float32) for _ in range(NG)],
            pltpu.VMEM_SHARED((npad, half), jnp.float32),
            [pltpu.SemaphoreType.DMA for _ in range(2)],
            [pltpu.SemaphoreType.DMA for _ in range(NG)],
            [pltpu.SemaphoreType.DMA for _ in range(NG)],
        ],
    )
    gcn = gcn_call(xws.reshape(2 * n, half), idx_pack, zeros2)

    out = pl.pallas_call(
        _combine_body,
        grid=(nb,),
        in_specs=[
            pl.BlockSpec((r, d), lambda i: (i, 0)),
            pl.BlockSpec((r, d), lambda i: (i, 0)),
            pl.BlockSpec((2, r, half), lambda i: (0, i, 0)),
            pl.BlockSpec((r, 1), lambda i: (i, 0)),
            pl.BlockSpec((r, 1), lambda i: (i, 0)),
            pl.BlockSpec((1, d), lambda i: (0, 0)),
        ],
        out_specs=pl.BlockSpec((r, d), lambda i: (i, 0)),
        out_shape=jax.ShapeDtypeStruct((n, d), jnp.float32),
    )(x, h2, gcn, degp0, degp1, bias.reshape(1, d))
    return out
